# baseline (speedup 1.0000x reference)
import jax
import jax.numpy as jnp
from jax import lax
from jax.experimental import pallas as pl
from jax.experimental.pallas import tpu as pltpu
from jax.experimental.pallas import tpu_sc as plsc

K = 100000
D = 64
NSENT = 4096
LS = 50

_info = plsc.get_sparse_core_info()
NC, NS, L = _info.num_cores, _info.num_subcores, _info.num_lanes
NW = NC * NS
S_PER_W = NSENT // NW
SENT_PER_SLOT = 4
N_SLOTS = S_PER_W // SENT_PER_SLOT


def _body(idx_hbm, table_hbm, out_hbm, idx_v, buf, obuf, gsem, ssem):
    wid = lax.axis_index("s") * NC + lax.axis_index("c")
    sent0 = wid * S_PER_W

    pltpu.sync_copy(idx_hbm.at[pl.ds(sent0, S_PER_W)], idx_v)

    def slot(t, carry):
        for j in range(SENT_PER_SLOT):
            pltpu.async_copy(
                table_hbm.at[idx_v.at[t * SENT_PER_SLOT + j]],
                buf.at[j],
                gsem,
            )
        for j in range(SENT_PER_SLOT):
            pltpu.make_async_copy(
                table_hbm.at[idx_v.at[t * SENT_PER_SLOT + j]],
                buf.at[j],
                gsem,
            ).wait()

        @plsc.parallel_loop(0, LS, step=2)
        def _relu_rows(i):
            for j in range(SENT_PER_SLOT):
                for r in range(2):
                    for k in range(D // L):
                        s = pl.ds(k * L, L)
                        obuf[j, i + r, s] = jnp.maximum(buf[j, i + r, s], 0.0)

        pltpu.sync_copy(
            obuf,
            out_hbm.at[pl.ds(sent0 + t * SENT_PER_SLOT, SENT_PER_SLOT)],
        )
        return carry

    lax.fori_loop(0, N_SLOTS, slot, 0)


@jax.jit
def _run(tags, table128):
    mesh = plsc.VectorSubcoreMesh(core_axis_name="c", subcore_axis_name="s")
    return pl.kernel(
        _body,
        out_type=jax.ShapeDtypeStruct((NSENT, LS, D), jnp.float32),
        mesh=mesh,
        scratch_types=[
            pltpu.VMEM((S_PER_W, LS), jnp.int32),
            pltpu.VMEM((SENT_PER_SLOT, LS, 2 * D), jnp.float32),
            pltpu.VMEM((SENT_PER_SLOT, LS, D), jnp.float32),
            pltpu.SemaphoreType.DMA,
            pltpu.SemaphoreType.DMA,
        ],
        compiler_params=pltpu.CompilerParams(use_tc_tiling_on_sc=True),
    )(tags, table128)


def kernel(preprocessed_tags, embedding_weight):
    tags = preprocessed_tags.astype(jnp.int32)
    table128 = jnp.pad(embedding_weight, ((0, 0), (0, D)))
    return _run(tags, table128)
